# baseline (speedup 1.0000x reference)
import functools

import jax
import jax.numpy as jnp
from jax import lax
from jax.experimental import pallas as pl
from jax.experimental.pallas import tpu as pltpu
from jax.experimental.pallas import tpu_sc as plsc

_NC = 2
_NS = 16
_L = 16
_K = 128
_S = 2
_KK = _S * _K


def _project_halves(features, weight, rows_per_block=2000):
    n, f = features.shape
    m = weight.shape[1]
    half = m // 2

    def body(f_ref, w_ref, o_ref):
        o_ref[0] = jnp.dot(f_ref[...], w_ref[0],
                           preferred_element_type=jnp.float32)

    w_halves = jnp.swapaxes(weight.reshape(f, 2, half), 0, 1)
    return pl.pallas_call(
        body,
        grid=(2, n // rows_per_block),
        in_specs=[
            pl.BlockSpec((rows_per_block, f), lambda c, r: (r, 0)),
            pl.BlockSpec((1, f, half), lambda c, r: (c, 0, 0)),
        ],
        out_specs=pl.BlockSpec((1, rows_per_block, half), lambda c, r: (c, r, 0)),
        out_shape=jax.ShapeDtypeStruct((2, n, half), jnp.float32),
    )(features, w_halves)


@functools.lru_cache
def _make_spmm(n_x, n_out, half, e_pad):
    ept = e_pad // _NS
    n_steps = ept // _KK
    rpt = n_out // _NS
    q_per_row = half // _L
    assert n_steps % 2 == 0 and n_steps >= 4

    mesh = plsc.VectorSubcoreMesh(core_axis_name="c", subcore_axis_name="s")

    @functools.partial(
        pl.kernel,
        out_type=jax.ShapeDtypeStruct((2, n_out, half), jnp.float32),
        mesh=mesh,
        scratch_types=[
            pltpu.VMEM((2, 3 * _S, _K), jnp.int32),
            pltpu.VMEM((2, _KK, half), jnp.float32),
            pltpu.VMEM_SHARED((n_out, half), jnp.float32),
            pltpu.VMEM_SHARED((n_out, half), jnp.float32),
            pltpu.SemaphoreType.DMA,
            pltpu.SemaphoreType.DMA,
            pltpu.SemaphoreType.DMA,
            pltpu.SemaphoreType.DMA,
        ],
        compiler_params=pltpu.CompilerParams(use_tc_tiling_on_sc=False,
                                             needs_layout_passes=False),
    )
    def spmm(x_hbm, edata_hbm, times_hbm, out_hbm,
             ebuf, rows, acc_a, acc_b, sem_i0, sem_i1, sem_g0, sem_g1):
        c = lax.axis_index("c")
        s = lax.axis_index("s")
        sem_i = (sem_i0, sem_i1)
        sem_g = (sem_g0, sem_g1)

        row0 = s * rpt
        nfull = rpt // _KK
        rem = rpt % _KK
        cbase = s * n_steps

        pltpu.sync_copy(times_hbm, ebuf.at[0].at[0].at[pl.ds(0, _L)])
        times = ebuf[0, 0, pl.ds(0, _L)][0]

        def zero_acc(acc_sh):
            def zero_row(i, carry):
                for q in range(q_per_row):
                    rows[0, i, pl.ds(q * _L, _L)] = jnp.zeros((_L,),
                                                              jnp.float32)
                return carry
            lax.fori_loop(0, _KK, zero_row, 0)
            for b in range(nfull):
                pltpu.sync_copy(rows.at[0],
                                acc_sh.at[pl.ds(row0 + b * _KK, _KK)])
            if rem:
                pltpu.sync_copy(rows.at[0].at[pl.ds(0, rem)],
                                acc_sh.at[pl.ds(row0 + nfull * _KK, rem)])

        def issue_idx(j, p):
            pltpu.async_copy(edata_hbm.at[cbase + j], ebuf.at[p], sem_i[p])

        def wait_idx(p):
            pltpu.make_async_copy(edata_hbm.at[0], ebuf.at[p], sem_i[p]).wait()

        def scale_scatter(p, unrolled, acc_sh):
            for u in range(_S):
                rows_u = rows.at[p].at[pl.ds(u * _K, _K)]

                def burst_group(g, carry, _u=u, _rows_u=rows_u):
                    av_bits = ebuf[p, 2 * _S + _u, pl.ds(g * _L, _L)]
                    av = plsc.bitcast(av_bits, jnp.float32)
                    e0 = g * _L
                    for i in range(_L):
                        a = av[i]
                        for q in range(q_per_row):
                            sl = pl.ds(q * _L, _L)
                            _rows_u[e0 + i, sl] = _rows_u[e0 + i, sl] * a
                    return carry
                if unrolled:
                    lax.fori_loop(0, _K // _L, burst_group, 0, unroll=8)
                else:
                    lax.fori_loop(0, _K // _L, burst_group, 0)
                pltpu.sync_copy(rows_u, acc_sh.at[ebuf.at[p].at[_S + u]],
                                add=True)

        def pipeline(src_ref, acc_sh):
            def issue_gathers(p):
                for u in range(_S):
                    pltpu.async_copy(src_ref.at[ebuf.at[p].at[u]],
                                     rows.at[p].at[pl.ds(u * _K, _K)],
                                     sem_g[p])

            def wait_gathers(p):
                pltpu.make_async_copy(src_ref.at[pl.ds(0, _KK)], rows.at[p],
                                      sem_g[p]).wait()

            issue_idx(0, 0)
            wait_idx(0)
            issue_gathers(0)
            issue_idx(1, 1)

            def pair(jp, carry):
                j = 2 * jp
                wait_idx(1)
                issue_gathers(1)
                wait_gathers(0)
                scale_scatter(0, True, acc_sh)
                issue_idx(j + 2, 0)
                wait_idx(0)
                issue_gathers(0)
                wait_gathers(1)
                scale_scatter(1, True, acc_sh)
                issue_idx(j + 3, 1)
                return carry
            lax.fori_loop(0, n_steps // 2 - 1, pair, 0)

            wait_idx(1)
            issue_gathers(1)
            wait_gathers(0)
            scale_scatter(0, False, acc_sh)
            wait_gathers(1)
            scale_scatter(1, False, acc_sh)

        def writeback(acc_sh):
            for b in range(nfull):
                sl = pl.ds(row0 + b * _KK, _KK)
                pltpu.sync_copy(acc_sh.at[sl], out_hbm.at[c].at[sl])
            if rem:
                sl = pl.ds(row0 + nfull * _KK, rem)
                pltpu.sync_copy(acc_sh.at[sl], out_hbm.at[c].at[sl])

        for b in range(nfull):
            sl = pl.ds(row0 + b * _KK, _KK)
            pltpu.sync_copy(x_hbm.at[c].at[sl], acc_b.at[sl])
        if rem:
            sl = pl.ds(row0 + nfull * _KK, rem)
            pltpu.sync_copy(x_hbm.at[c].at[sl], acc_b.at[sl])
        plsc.subcore_barrier()

        def round_body(r, carry):
            odd = (r % 2) == 1

            @pl.when(odd)
            def _():
                zero_acc(acc_b)
                plsc.subcore_barrier()
                pipeline(acc_a, acc_b)
                plsc.subcore_barrier()

            @pl.when(jnp.logical_not(odd))
            def _():
                zero_acc(acc_a)
                plsc.subcore_barrier()
                pipeline(acc_b, acc_a)
                plsc.subcore_barrier()
            return carry
        lax.fori_loop(0, times, round_body, 0)

        @pl.when((times % 2) == 1)
        def _():
            writeback(acc_a)

        @pl.when((times % 2) == 0)
        def _():
            writeback(acc_b)

    return spmm


def kernel(features, adj_values, weight, edge_index, times):
    n, _ = features.shape
    m = weight.shape[1]
    half = m // 2
    e = edge_index.shape[1]

    src = edge_index[1].astype(jnp.int32)
    dst = edge_index[0].astype(jnp.int32)
    adj = adj_values.astype(jnp.float32)

    grain = _NS * 2 * _KK
    e_pad = ((e + grain - 1) // grain) * grain
    if e_pad != e:
        pad = e_pad - e
        src = jnp.concatenate([src, jnp.zeros((pad,), jnp.int32)])
        dst = jnp.concatenate([dst, jnp.zeros((pad,), jnp.int32)])
        adj = jnp.concatenate([adj, jnp.zeros((pad,), jnp.float32)])

    n_steps = e_pad // (_NS * _KK)
    adj_bits = lax.bitcast_convert_type(adj, jnp.int32)
    edata = jnp.stack([src, dst, adj_bits])
    edata = edata.reshape(3, _NS, n_steps, _S, _K)
    edata = jnp.transpose(edata, (1, 2, 0, 3, 4))
    edata = edata.reshape(_NS * n_steps, 3 * _S, _K)

    times_v = jnp.full((_L,), 1, jnp.int32) * times

    row_grain = _NS * 8
    n_pad = ((n + row_grain - 1) // row_grain) * row_grain

    features_p = jnp.pad(features, ((0, n_pad - n), (0, 0)))
    support2 = _project_halves(features_p, weight,
                               rows_per_block=n_pad // 8)
    spmm = _make_spmm(n_pad, n_pad, half, e_pad)
    out2 = spmm(support2, edata, times_v)
    return jnp.swapaxes(out2[:, :n, :], 0, 1).reshape(n, m)


# speedup vs baseline: 1.0383x; 1.0383x over previous
import functools

import jax
import jax.numpy as jnp
from jax import lax
from jax.experimental import pallas as pl
from jax.experimental.pallas import tpu as pltpu
from jax.experimental.pallas import tpu_sc as plsc

_NC = 2
_NS = 16
_L = 16
_K = 128


def _project_halves(features, weight, rows_per_block=2000):
    n, f = features.shape
    m = weight.shape[1]
    half = m // 2

    def body(f_ref, w_ref, o_ref):
        o_ref[0] = jnp.dot(f_ref[...], w_ref[0],
                           preferred_element_type=jnp.float32)

    w_halves = jnp.swapaxes(weight.reshape(f, 2, half), 0, 1)
    return pl.pallas_call(
        body,
        grid=(2, n // rows_per_block),
        in_specs=[
            pl.BlockSpec((rows_per_block, f), lambda c, r: (r, 0)),
            pl.BlockSpec((1, f, half), lambda c, r: (c, 0, 0)),
        ],
        out_specs=pl.BlockSpec((1, rows_per_block, half), lambda c, r: (c, r, 0)),
        out_shape=jax.ShapeDtypeStruct((2, n, half), jnp.float32),
    )(features, w_halves)


@functools.lru_cache
def _make_spmm(n_x, n_out, half, e_pad):
    ept = e_pad // _NS
    n_chunks = ept // _K
    rpt = n_out // _NS
    q_per_row = half // _L
    assert n_chunks % 2 == 0 and n_chunks >= 4

    mesh = plsc.VectorSubcoreMesh(core_axis_name="c", subcore_axis_name="s")

    @functools.partial(
        pl.kernel,
        out_type=jax.ShapeDtypeStruct((2, n_out, half), jnp.float32),
        mesh=mesh,
        scratch_types=[
            pltpu.VMEM((2, 3, _K), jnp.int32),
            pltpu.VMEM((2, _K, half), jnp.float32),
            pltpu.VMEM_SHARED((n_out, half), jnp.float32),
            pltpu.VMEM_SHARED((n_out, half), jnp.float32),
            pltpu.SemaphoreType.DMA,
            pltpu.SemaphoreType.DMA,
            pltpu.SemaphoreType.DMA,
            pltpu.SemaphoreType.DMA,
        ],
        compiler_params=pltpu.CompilerParams(use_tc_tiling_on_sc=False,
                                             needs_layout_passes=False),
    )
    def spmm(x_hbm, edata_hbm, times_hbm, out_hbm,
             ebuf, rows, acc_a, acc_b, sem_i0, sem_i1, sem_g0, sem_g1):
        c = lax.axis_index("c")
        s = lax.axis_index("s")
        sem_i = (sem_i0, sem_i1)
        sem_g = (sem_g0, sem_g1)

        row0 = s * rpt
        nfull = rpt // _K
        rem = rpt % _K
        cbase = s * n_chunks

        pltpu.sync_copy(times_hbm, ebuf.at[0].at[0].at[pl.ds(0, _L)])
        times = ebuf[0, 0, pl.ds(0, _L)][0]

        def zero_acc(acc_sh):
            def zero_row(i, carry):
                for q in range(q_per_row):
                    rows[0, i, pl.ds(q * _L, _L)] = jnp.zeros((_L,),
                                                              jnp.float32)
                return carry
            lax.fori_loop(0, _K, zero_row, 0)
            for b in range(nfull):
                pltpu.sync_copy(rows.at[0],
                                acc_sh.at[pl.ds(row0 + b * _K, _K)])
            if rem:
                pltpu.sync_copy(rows.at[0].at[pl.ds(0, rem)],
                                acc_sh.at[pl.ds(row0 + nfull * _K, rem)])

        def issue_idx(j, p):
            pltpu.async_copy(edata_hbm.at[cbase + j], ebuf.at[p], sem_i[p])

        def wait_idx(p):
            pltpu.make_async_copy(edata_hbm.at[0], ebuf.at[p], sem_i[p]).wait()

        def scale_scatter(p, unrolled, acc_sh):
            rows_p = rows.at[p]

            def group(g, carry):
                av_bits = ebuf[p, 2, pl.ds(g * _L, _L)]
                av = plsc.bitcast(av_bits, jnp.float32)
                e0 = g * _L
                for i in range(_L):
                    a = av[i]
                    for q in range(q_per_row):
                        sl = pl.ds(q * _L, _L)
                        rows_p[e0 + i, sl] = rows_p[e0 + i, sl] * a
                return carry
            if unrolled:
                for g in range(_K // _L):
                    group(g, 0)
            else:
                lax.fori_loop(0, _K // _L, group, 0)
            pltpu.sync_copy(rows_p, acc_sh.at[ebuf.at[p].at[1]], add=True)

        def pipeline(src0_ref, src1_ref, acc_sh):
            srcs = (src0_ref, src1_ref)

            def issue_gather(p):
                pltpu.async_copy(srcs[p].at[ebuf.at[p].at[0]], rows.at[p],
                                 sem_g[p])

            def wait_gather(p):
                pltpu.make_async_copy(srcs[p].at[pl.ds(0, _K)], rows.at[p],
                                      sem_g[p]).wait()

            issue_idx(0, 0)
            wait_idx(0)
            issue_gather(0)
            issue_idx(1, 1)

            def pair(jp, carry):
                j = 2 * jp
                wait_idx(1)
                issue_gather(1)
                wait_gather(0)
                scale_scatter(0, True, acc_sh)
                issue_idx(j + 2, 0)
                wait_idx(0)
                issue_gather(0)
                wait_gather(1)
                scale_scatter(1, True, acc_sh)
                issue_idx(j + 3, 1)
                return carry
            lax.fori_loop(0, n_chunks // 2 - 1, pair, 0)

            wait_idx(1)
            issue_gather(1)
            wait_gather(0)
            scale_scatter(0, False, acc_sh)
            wait_gather(1)
            scale_scatter(1, False, acc_sh)

        def writeback(acc_sh):
            for b in range(nfull):
                sl = pl.ds(row0 + b * _K, _K)
                pltpu.sync_copy(acc_sh.at[sl], out_hbm.at[c].at[sl])
            if rem:
                sl = pl.ds(row0 + nfull * _K, rem)
                pltpu.sync_copy(acc_sh.at[sl], out_hbm.at[c].at[sl])

        zero_acc(acc_a)
        plsc.subcore_barrier()
        pipeline(x_hbm.at[c], x_hbm.at[c], acc_a)
        plsc.subcore_barrier()
        writeback(acc_a)
        plsc.subcore_barrier()

        def round_body(r, carry):
            odd = (r % 2) == 1

            @pl.when(odd)
            def _():
                zero_acc(acc_b)
                plsc.subcore_barrier()
                pipeline(out_hbm.at[c], acc_a, acc_b)
                plsc.subcore_barrier()
                writeback(acc_b)
                plsc.subcore_barrier()

            @pl.when(jnp.logical_not(odd))
            def _():
                zero_acc(acc_a)
                plsc.subcore_barrier()
                pipeline(out_hbm.at[c], acc_b, acc_a)
                plsc.subcore_barrier()
                writeback(acc_a)
                plsc.subcore_barrier()
            return carry
        lax.fori_loop(1, times, round_body, 0)

    return spmm


def kernel(features, adj_values, weight, edge_index, times):
    n, _ = features.shape
    m = weight.shape[1]
    half = m // 2
    e = edge_index.shape[1]

    src = edge_index[1].astype(jnp.int32)
    dst = edge_index[0].astype(jnp.int32)
    adj = adj_values.astype(jnp.float32)

    grain = _NS * 2 * _K
    e_pad = ((e + grain - 1) // grain) * grain
    if e_pad != e:
        pad = e_pad - e
        src = jnp.concatenate([src, jnp.zeros((pad,), jnp.int32)])
        dst = jnp.concatenate([dst, jnp.zeros((pad,), jnp.int32)])
        adj = jnp.concatenate([adj, jnp.zeros((pad,), jnp.float32)])

    n_chunks = e_pad // (_NS * _K)
    adj_bits = lax.bitcast_convert_type(adj, jnp.int32)
    edata = jnp.stack([src, dst, adj_bits])
    edata = edata.reshape(3, _NS, n_chunks, _K)
    edata = jnp.transpose(edata, (1, 2, 0, 3)).reshape(_NS * n_chunks, 3, _K)

    times_v = jnp.full((_L,), 1, jnp.int32) * times

    row_grain = _NS * 8
    n_pad = ((n + row_grain - 1) // row_grain) * row_grain

    support2 = _project_halves(features, weight)
    spmm = _make_spmm(n, n_pad, half, e_pad)
    out2 = spmm(support2, edata, times_v)
    return jnp.swapaxes(out2[:, :n, :], 0, 1).reshape(n, m)


# speedup vs baseline: 1.1095x; 1.0685x over previous
import functools

import jax
import jax.numpy as jnp
from jax import lax
from jax.experimental import pallas as pl
from jax.experimental.pallas import tpu as pltpu
from jax.experimental.pallas import tpu_sc as plsc

_NC = 2
_NS = 16
_L = 16
_K = 128


def _project_halves(features, weight, rows_per_block=2000):
    n, f = features.shape
    m = weight.shape[1]
    half = m // 2

    def body(f_ref, w_ref, o_ref):
        o_ref[0] = jnp.dot(f_ref[...], w_ref[0],
                           preferred_element_type=jnp.float32)

    w_halves = jnp.swapaxes(weight.reshape(f, 2, half), 0, 1)
    return pl.pallas_call(
        body,
        grid=(2, n // rows_per_block),
        in_specs=[
            pl.BlockSpec((rows_per_block, f), lambda c, r: (r, 0)),
            pl.BlockSpec((1, f, half), lambda c, r: (c, 0, 0)),
        ],
        out_specs=pl.BlockSpec((1, rows_per_block, half), lambda c, r: (c, r, 0)),
        out_shape=jax.ShapeDtypeStruct((2, n, half), jnp.float32),
    )(features, w_halves)


@functools.lru_cache
def _make_spmm(n_x, n_out, half, e_pad):
    ept = e_pad // _NS
    n_chunks = ept // _K
    rpt = n_out // _NS
    q_per_row = half // _L
    assert n_chunks % 2 == 0 and n_chunks >= 4

    mesh = plsc.VectorSubcoreMesh(core_axis_name="c", subcore_axis_name="s")

    @functools.partial(
        pl.kernel,
        out_type=jax.ShapeDtypeStruct((2, n_out, half), jnp.float32),
        mesh=mesh,
        scratch_types=[
            pltpu.VMEM((2, 3, _K), jnp.int32),
            pltpu.VMEM((2, _K, half), jnp.float32),
            pltpu.VMEM_SHARED((n_out, half), jnp.float32),
            pltpu.VMEM_SHARED((n_out, half), jnp.float32),
            pltpu.SemaphoreType.DMA,
            pltpu.SemaphoreType.DMA,
            pltpu.SemaphoreType.DMA,
            pltpu.SemaphoreType.DMA,
        ],
        compiler_params=pltpu.CompilerParams(use_tc_tiling_on_sc=False,
                                             needs_layout_passes=False),
    )
    def spmm(x_hbm, edata_hbm, times_hbm, out_hbm,
             ebuf, rows, acc_a, acc_b, sem_i0, sem_i1, sem_g0, sem_g1):
        c = lax.axis_index("c")
        s = lax.axis_index("s")
        sem_i = (sem_i0, sem_i1)
        sem_g = (sem_g0, sem_g1)

        row0 = s * rpt
        nfull = rpt // _K
        rem = rpt % _K
        cbase = s * n_chunks

        pltpu.sync_copy(times_hbm, ebuf.at[0].at[0].at[pl.ds(0, _L)])
        times = ebuf[0, 0, pl.ds(0, _L)][0]

        def zero_acc(acc_sh):
            def zero_row(i, carry):
                for q in range(q_per_row):
                    rows[0, i, pl.ds(q * _L, _L)] = jnp.zeros((_L,),
                                                              jnp.float32)
                return carry
            lax.fori_loop(0, _K, zero_row, 0)
            for b in range(nfull):
                pltpu.sync_copy(rows.at[0],
                                acc_sh.at[pl.ds(row0 + b * _K, _K)])
            if rem:
                pltpu.sync_copy(rows.at[0].at[pl.ds(0, rem)],
                                acc_sh.at[pl.ds(row0 + nfull * _K, rem)])

        def issue_idx(j, p):
            pltpu.async_copy(edata_hbm.at[cbase + j], ebuf.at[p], sem_i[p])

        def wait_idx(p):
            pltpu.make_async_copy(edata_hbm.at[0], ebuf.at[p], sem_i[p]).wait()

        def scale_scatter(p, unrolled, acc_sh):
            rows_p = rows.at[p]

            def group(g, carry):
                av_bits = ebuf[p, 2, pl.ds(g * _L, _L)]
                av = plsc.bitcast(av_bits, jnp.float32)
                e0 = g * _L
                for i in range(_L):
                    a = av[i]
                    for q in range(q_per_row):
                        sl = pl.ds(q * _L, _L)
                        rows_p[e0 + i, sl] = rows_p[e0 + i, sl] * a
                return carry
            if unrolled:
                for g in range(_K // _L):
                    group(g, 0)
            else:
                lax.fori_loop(0, _K // _L, group, 0)
            pltpu.sync_copy(rows_p, acc_sh.at[ebuf.at[p].at[1]], add=True)

        def pipeline(src0_ref, src1_ref, acc_sh):
            srcs = (src0_ref, src1_ref)

            def issue_gather(p):
                pltpu.async_copy(srcs[p].at[ebuf.at[p].at[0]], rows.at[p],
                                 sem_g[p])

            def wait_gather(p):
                pltpu.make_async_copy(srcs[p].at[pl.ds(0, _K)], rows.at[p],
                                      sem_g[p]).wait()

            issue_idx(0, 0)
            wait_idx(0)
            issue_gather(0)
            issue_idx(1, 1)

            def pair(jp, carry):
                j = 2 * jp
                wait_idx(1)
                issue_gather(1)
                wait_gather(0)
                scale_scatter(0, True, acc_sh)
                issue_idx(j + 2, 0)
                wait_idx(0)
                issue_gather(0)
                wait_gather(1)
                scale_scatter(1, True, acc_sh)
                issue_idx(j + 3, 1)
                return carry
            lax.fori_loop(0, n_chunks // 2 - 1, pair, 0)

            wait_idx(1)
            issue_gather(1)
            wait_gather(0)
            scale_scatter(0, False, acc_sh)
            wait_gather(1)
            scale_scatter(1, False, acc_sh)

        def writeback(acc_sh):
            for b in range(nfull):
                sl = pl.ds(row0 + b * _K, _K)
                pltpu.sync_copy(acc_sh.at[sl], out_hbm.at[c].at[sl])
            if rem:
                sl = pl.ds(row0 + nfull * _K, rem)
                pltpu.sync_copy(acc_sh.at[sl], out_hbm.at[c].at[sl])

        for b in range(nfull):
            sl = pl.ds(row0 + b * _K, _K)
            pltpu.sync_copy(x_hbm.at[c].at[sl], acc_b.at[sl])
        if rem:
            sl = pl.ds(row0 + nfull * _K, rem)
            pltpu.sync_copy(x_hbm.at[c].at[sl], acc_b.at[sl])
        zero_acc(acc_a)
        plsc.subcore_barrier()
        pipeline(x_hbm.at[c], acc_b, acc_a)
        plsc.subcore_barrier()

        def round_body(r, carry):
            odd = (r % 2) == 1

            @pl.when(odd)
            def _():
                zero_acc(acc_b)
                plsc.subcore_barrier()
                pipeline(acc_a, acc_a, acc_b)
                plsc.subcore_barrier()

            @pl.when(jnp.logical_not(odd))
            def _():
                zero_acc(acc_a)
                plsc.subcore_barrier()
                pipeline(acc_b, acc_b, acc_a)
                plsc.subcore_barrier()
            return carry
        lax.fori_loop(1, times, round_body, 0)

        @pl.when((times % 2) == 1)
        def _():
            writeback(acc_a)

        @pl.when((times % 2) == 0)
        def _():
            writeback(acc_b)

    return spmm


def kernel(features, adj_values, weight, edge_index, times):
    n, _ = features.shape
    m = weight.shape[1]
    half = m // 2
    e = edge_index.shape[1]

    src = edge_index[1].astype(jnp.int32)
    dst = edge_index[0].astype(jnp.int32)
    adj = adj_values.astype(jnp.float32)

    grain = _NS * 2 * _K
    e_pad = ((e + grain - 1) // grain) * grain
    if e_pad != e:
        pad = e_pad - e
        src = jnp.concatenate([src, jnp.zeros((pad,), jnp.int32)])
        dst = jnp.concatenate([dst, jnp.zeros((pad,), jnp.int32)])
        adj = jnp.concatenate([adj, jnp.zeros((pad,), jnp.float32)])

    n_chunks = e_pad // (_NS * _K)
    adj_bits = lax.bitcast_convert_type(adj, jnp.int32)
    edata = jnp.stack([src, dst, adj_bits])
    edata = edata.reshape(3, _NS, n_chunks, _K)
    edata = jnp.transpose(edata, (1, 2, 0, 3)).reshape(_NS * n_chunks, 3, _K)

    times_v = jnp.full((_L,), 1, jnp.int32) * times

    row_grain = _NS * 8
    n_pad = ((n + row_grain - 1) // row_grain) * row_grain

    features_p = jnp.pad(features, ((0, n_pad - n), (0, 0)))
    support2 = _project_halves(features_p, weight,
                               rows_per_block=n_pad // 8)
    spmm = _make_spmm(n_pad, n_pad, half, e_pad)
    out2 = spmm(support2, edata, times_v)
    return jnp.swapaxes(out2[:, :n, :], 0, 1).reshape(n, m)
